# Initial kernel scaffold; baseline (speedup 1.0000x reference)
import jax
import jax.numpy as jnp
from jax.experimental import pallas as pl


def kernel(x):
    raise NotImplementedError("write your pallas kernel here")


# speedup vs baseline: 1.8237x; 1.8237x over previous
import math

import jax
import jax.numpy as jnp
import numpy as np
from jax.experimental import pallas as pl

_T = 2048
_PRED = 96
_K = 8
_F = 1024

_k = np.arange(1, _F + 1, dtype=np.int64)
_t = np.arange(_T, dtype=np.int64)
_ang = (2.0 * math.pi / _T) * ((_k[:, None] * _t[None, :]) % _T)
_CM = np.cos(_ang).astype(np.float32)
_SM = np.sin(_ang).astype(np.float32)


def _seasonality_kernel(x_ref, cm_ref, sm_ref, out_ref):
    xb = x_ref[0]
    cm = cm_ref[...]
    sm = sm_ref[...]
    dot = lambda a, b, dn: jax.lax.dot_general(
        a, b, dimension_numbers=(dn, ((), ())),
        preferred_element_type=jnp.float32,
        precision=jax.lax.Precision.HIGHEST)
    p = dot(cm, xb, ((1,), (0,)))
    q = dot(sm, xb, ((1,), (0,)))
    mag = p * p + q * q
    row = jax.lax.broadcasted_iota(jnp.int32, mag.shape, 0)
    vals = jnp.where(row == _F - 1, -1.0, mag)
    sel = jnp.zeros(mag.shape, dtype=jnp.bool_)
    for _ in range(_K):
        m = jnp.max(vals, axis=0, keepdims=True)
        eligible = vals == m
        idx = jnp.min(jnp.where(eligible, row, _F), axis=0, keepdims=True)
        onehot = row == idx
        sel = jnp.logical_or(sel, onehot)
        vals = jnp.where(onehot, -2.0, vals)
    c = 2.0 / _T
    a = jnp.where(sel, p * c, 0.0)
    b = jnp.where(sel, q * c, 0.0)
    head = dot(cm, a, ((0,), (0,))) + dot(sm, b, ((0,), (0,)))
    out_ref[0, :_T, :] = head
    out_ref[0, _T:, :] = head[:_PRED, :]


def _impl(x):
    bsz, t, d = x.shape
    cm = jnp.asarray(_CM)
    sm = jnp.asarray(_SM)
    return pl.pallas_call(
        _seasonality_kernel,
        grid=(bsz,),
        in_specs=[
            pl.BlockSpec((1, t, d), lambda i: (i, 0, 0)),
            pl.BlockSpec((_F, _T), lambda i: (0, 0)),
            pl.BlockSpec((_F, _T), lambda i: (0, 0)),
        ],
        out_specs=pl.BlockSpec((1, t + _PRED, d), lambda i: (i, 0, 0)),
        out_shape=jax.ShapeDtypeStruct((bsz, t + _PRED, d), jnp.float32),
    )(x, cm, sm)


def kernel(x):
    return _impl(x)
